# baseline (speedup 1.0000x reference)
import jax
import jax.numpy as jnp
from jax import lax
from jax.experimental import pallas as pl
from jax.experimental.pallas import tpu as pltpu
from jax.experimental.pallas import tpu_sc as plsc

_N = 10000
_E = 320000
_D = 128
_G = 64
_EPS = 1e-5

_NC = 2
_NS = 16
_NW = _NC * _NS
_CHUNK = 128
_EPT = 10240
_CH = _EPT // _CHUNK
_NIDX = 2
_CHH = _CH // _NIDX
_NBUF = 2
_E_PAD = _NW * _EPT
_ACC_ROWS = 10240
_ZPT = _ACC_ROWS // _NS
_RPT = _N // _NS


def _sc_degree_body(dst_hbm, out_hbm, dst_v, ones_v, zv, acc):
    c = lax.axis_index("c")
    s = lax.axis_index("s")
    wid = s * _NC + c

    @pl.loop(0, _ZPT // 16)
    def _(i):
        zv[pl.ds(i * 16, 16)] = jnp.zeros((16,), jnp.float32)

    @pl.loop(0, _CHUNK // 16)
    def _(i):
        ones_v[pl.ds(i * 16, 16)] = jnp.ones((16,), jnp.float32)

    pltpu.sync_copy(zv, acc.at[pl.ds(s * _ZPT, _ZPT)])
    plsc.subcore_barrier()

    pltpu.sync_copy(dst_hbm.at[wid], dst_v)

    @pl.loop(0, _CH)
    def _(j):
        pltpu.sync_copy(ones_v, acc.at[dst_v.at[j]], add=True)

    plsc.subcore_barrier()
    pltpu.sync_copy(acc.at[pl.ds(s * _ZPT, _ZPT)],
                    out_hbm.at[c, pl.ds(s * _ZPT, _ZPT)])


def _sc_scatter_body(g_hbm, src_hbm, dst_hbm, out_hbm,
                     src_v, dst_v, rows0, rows1, acc,
                     gsem0, gsem1, ssem0, ssem1):
    c = lax.axis_index("c")
    s = lax.axis_index("s")
    wid = s * _NC + c
    rows = (rows0, rows1)
    gsem = (gsem0, gsem1)
    ssem = (ssem0, ssem1)

    @pl.loop(0, _CHUNK)
    def _(r):
        for c8 in range(8):
            rows0[r, pl.ds(c8 * 16, 16)] = jnp.zeros((16,), jnp.float32)

    @pl.loop(0, _ZPT // _CHUNK)
    def _(i):
        pltpu.sync_copy(rows0, acc.at[pl.ds(s * _ZPT + i * _CHUNK, _CHUNK), :])

    plsc.subcore_barrier()

    def gather_start(j, b):
        pltpu.async_copy(g_hbm.at[src_v.at[j]], rows[b], gsem[b])

    def gather_wait(j, b):
        pltpu.make_async_copy(g_hbm.at[src_v.at[j]], rows[b], gsem[b]).wait()

    def scat_start(j, b):
        pltpu.async_copy(rows[b], acc.at[dst_v.at[j]], ssem[b], add=True)

    def scat_wait(j, b):
        pltpu.make_async_copy(rows[b], acc.at[dst_v.at[j]], ssem[b]).wait()

    for h in range(_NIDX):
        pltpu.sync_copy(src_hbm.at[wid, pl.ds(h * _CHH, _CHH)], src_v)
        pltpu.sync_copy(dst_hbm.at[wid, pl.ds(h * _CHH, _CHH)], dst_v)

        for b in range(_NBUF - 1):
            gather_start(b, b)

        @pl.loop(0, _CHH, step=_NBUF)
        def _(jb):
            for b in range(_NBUF):
                j = jb + b
                gather_wait(j, b)
                scat_start(j, b)
                bn = (b + _NBUF - 1) % _NBUF

                if b == 0:
                    @pl.when(jb > 0)
                    def _():
                        scat_wait(jb - 1, bn)
                else:
                    scat_wait(j - 1, bn)

                @pl.when(j + _NBUF - 1 < _CHH)
                def _():
                    gather_start(j + _NBUF - 1, bn)

        scat_wait(_CHH - 1, (_CHH - 1) % _NBUF)

    plsc.subcore_barrier()
    pltpu.sync_copy(acc.at[pl.ds(s * _ZPT, _ZPT), :],
                    out_hbm.at[c, pl.ds(s * _ZPT, _ZPT), :])


def _sc_mesh():
    return plsc.VectorSubcoreMesh(core_axis_name="c", subcore_axis_name="s",
                                  num_cores=_NC, num_subcores=_NS)


def _sc_degree(dst_r):
    return pl.kernel(
        _sc_degree_body,
        out_type=jax.ShapeDtypeStruct((_NC, _ACC_ROWS), jnp.float32),
        mesh=_sc_mesh(),
        scratch_types=[
            pltpu.VMEM((_CH, _CHUNK), jnp.int32),
            pltpu.VMEM((_CHUNK,), jnp.float32),
            pltpu.VMEM((_ZPT,), jnp.float32),
            pltpu.VMEM_SHARED((_ACC_ROWS,), jnp.float32),
        ],
    )(dst_r)


def _sc_scatter(g, src_r, dst_r):
    return pl.kernel(
        _sc_scatter_body,
        out_type=jax.ShapeDtypeStruct((_NC, _ACC_ROWS, _D), jnp.float32),
        mesh=_sc_mesh(),
        scratch_types=(
            [pltpu.VMEM((_CHH, _CHUNK), jnp.int32)] * 2
            + [pltpu.VMEM((_CHUNK, _D), jnp.float32)] * _NBUF
            + [pltpu.VMEM_SHARED((_ACC_ROWS, _D), jnp.float32)]
            + [pltpu.SemaphoreType.DMA] * (2 * _NBUF)
        ),
    )(g, src_r, dst_r)


_BR = 1000


def _tc1_body(x_ref, w_ref, da_ref, db_ref, g_ref, dis_ref):
    deg = da_ref[...] + db_ref[...] + 1.0
    dis = lax.rsqrt(deg)
    dis_ref[...] = dis
    g_ref[...] = dis * jnp.dot(x_ref[...], w_ref[...],
                               preferred_element_type=jnp.float32)


def _tc_mid_body(pa, pb, g, dis, b, gm, bt, mn, vr, w, out):
    conv = dis[...] * (pa[0] + pb[0] + g[...]) + b[...]
    h = (conv - mn[...]) * lax.rsqrt(vr[...] + _EPS) * gm[...] + bt[...]
    h = jnp.maximum(h, 0.0)
    out[...] = dis[...] * jnp.dot(h, w[...],
                                  preferred_element_type=jnp.float32)


def _tc_fin_body(pa, pb, g, dis, b, gm, bt, mn, vr, oh_ref,
                 h_ref, pool_ref):
    i = pl.program_id(0)
    conv = dis[...] * (pa[0] + pb[0] + g[...]) + b[...]
    h = (conv - mn[...]) * lax.rsqrt(vr[...] + _EPS) * gm[...] + bt[...]
    h = jnp.maximum(h, 0.0)
    h_ref[...] = h
    part = lax.dot_general(oh_ref[...], h, (((0,), (0,)), ((), ())),
                           preferred_element_type=jnp.float32)

    @pl.when(i == 0)
    def _():
        pool_ref[...] = part

    @pl.when(i != 0)
    def _():
        pool_ref[...] += part


def _rows_spec():
    return pl.BlockSpec((_BR, _D), lambda i: (i, 0))


def _col_spec():
    return pl.BlockSpec((_BR, 1), lambda i: (i, 0))


def _param_spec():
    return pl.BlockSpec((1, _D), lambda i: (0, 0))


def _full_spec():
    return pl.BlockSpec((_D, _D), lambda i: (0, 0))


def _tc1(x, W0, da, db):
    return pl.pallas_call(
        _tc1_body,
        grid=(_N // _BR,),
        in_specs=[_rows_spec(), _full_spec(), _col_spec(), _col_spec()],
        out_specs=[_rows_spec(), _col_spec()],
        out_shape=[jax.ShapeDtypeStruct((_N, _D), jnp.float32),
                   jax.ShapeDtypeStruct((_N, 1), jnp.float32)],
    )(x, W0, da, db)


def _part_spec(core):
    return pl.BlockSpec((1, _BR, _D), lambda i, c=core: (c, i, 0))


def _tc_mid(p, g, dis, b, gm, bt, mn, vr, W):
    return pl.pallas_call(
        _tc_mid_body,
        grid=(_N // _BR,),
        in_specs=[_part_spec(0), _part_spec(1), _rows_spec(), _col_spec(),
                  _param_spec(), _param_spec(), _param_spec(),
                  _param_spec(), _param_spec(), _full_spec()],
        out_specs=_rows_spec(),
        out_shape=jax.ShapeDtypeStruct((_N, _D), jnp.float32),
    )(p, p, g, dis, b, gm, bt, mn, vr, W)


def _tc_fin(p, g, dis, b, gm, bt, mn, vr, onehot):
    return pl.pallas_call(
        _tc_fin_body,
        grid=(_N // _BR,),
        in_specs=[_part_spec(0), _part_spec(1), _rows_spec(), _col_spec(),
                  _param_spec(), _param_spec(), _param_spec(),
                  _param_spec(), _param_spec(),
                  pl.BlockSpec((_BR, _G), lambda i: (i, 0))],
        out_specs=[_rows_spec(), pl.BlockSpec((_G, _D), lambda i: (0, 0))],
        out_shape=[jax.ShapeDtypeStruct((_N, _D), jnp.float32),
                   jax.ShapeDtypeStruct((_G, _D), jnp.float32)],
    )(p, p, g, dis, b, gm, bt, mn, vr, onehot)


def kernel(x, edge_index, batch, W0, b0, bn0_gamma, bn0_beta, bn0_mean,
           bn0_var, W1, b1, bn1_gamma, bn1_beta, bn1_mean, bn1_var):
    pad = _E_PAD - _E
    pad_ids = jnp.arange(pad, dtype=jnp.int32)
    src_r = jnp.concatenate(
        [edge_index[0], pad_ids % _N]).reshape(_NW, _CH, _CHUNK)
    dst_r = jnp.concatenate(
        [edge_index[1], _N + pad_ids % (_ACC_ROWS - _N)]).reshape(_NW, _CH, _CHUNK)

    degp = _sc_degree(dst_r)
    da = degp[0, :_N, None]
    db = degp[1, :_N, None]

    onehot = (batch[:, None] ==
              jnp.arange(_G, dtype=jnp.int32)[None, :]).astype(jnp.float32)

    g0, dis = _tc1(x, W0, da, db)
    p0 = _sc_scatter(g0, src_r, dst_r)
    g1 = _tc_mid(p0, g0, dis,
                 b0[None, :], bn0_gamma[None, :], bn0_beta[None, :],
                 bn0_mean[None, :], bn0_var[None, :], W1)
    p1 = _sc_scatter(g1, src_r, dst_r)
    h1, pool = _tc_fin(p1, g1, dis,
                       b1[None, :], bn1_gamma[None, :], bn1_beta[None, :],
                       bn1_mean[None, :], bn1_var[None, :], onehot)
    return h1, pool


# speedup vs baseline: 1.1393x; 1.1393x over previous
import jax
import jax.numpy as jnp
from jax import lax
from jax.experimental import pallas as pl
from jax.experimental.pallas import tpu as pltpu
from jax.experimental.pallas import tpu_sc as plsc

_N = 10000
_E = 320000
_D = 128
_G = 64
_EPS = 1e-5

_NC = 2
_NS = 16
_NW = _NC * _NS
_CHUNK = 128
_EPT = 10240
_CH = _EPT // _CHUNK
_NIDX = 2
_CHH = _CH // _NIDX
_NBUF = 2
_E_PAD = _NW * _EPT
_ACC_ROWS = 10240
_ZPT = _ACC_ROWS // _NS
_RPT = _N // _NS


def _sc_degree_body(dst_hbm, out_hbm, dst_v, ones_v, zv, acc):
    c = lax.axis_index("c")
    s = lax.axis_index("s")
    wid = s * _NC + c

    @pl.loop(0, _ZPT // 16)
    def _(i):
        zv[pl.ds(i * 16, 16)] = jnp.zeros((16,), jnp.float32)

    @pl.loop(0, _CHUNK // 16)
    def _(i):
        ones_v[pl.ds(i * 16, 16)] = jnp.ones((16,), jnp.float32)

    pltpu.sync_copy(zv, acc.at[pl.ds(s * _ZPT, _ZPT)])
    plsc.subcore_barrier()

    pltpu.sync_copy(dst_hbm.at[wid], dst_v)

    @pl.loop(0, _CH)
    def _(j):
        pltpu.sync_copy(ones_v, acc.at[dst_v.at[j]], add=True)

    plsc.subcore_barrier()
    pltpu.sync_copy(acc.at[pl.ds(s * _ZPT, _ZPT)],
                    out_hbm.at[c, pl.ds(s * _ZPT, _ZPT)])


def _sc_scatter_body(g_hbm, src_hbm, dst_hbm, out_hbm,
                     src_v, dst_v, rows0, rows1, acc,
                     gsem0, gsem1, ssem0, ssem1):
    c = lax.axis_index("c")
    s = lax.axis_index("s")
    wid = s * _NC + c
    rows = (rows0, rows1)
    gsem = (gsem0, gsem1)
    ssem = (ssem0, ssem1)

    @pl.loop(0, _CHUNK)
    def _(r):
        for c8 in range(8):
            rows0[r, pl.ds(c8 * 16, 16)] = jnp.zeros((16,), jnp.float32)

    @pl.loop(0, _ZPT // _CHUNK)
    def _(i):
        pltpu.sync_copy(rows0, acc.at[pl.ds(s * _ZPT + i * _CHUNK, _CHUNK), :])

    plsc.subcore_barrier()

    def gather_start(j, b):
        pltpu.async_copy(g_hbm.at[src_v.at[j]], rows[b], gsem[b])

    def gather_wait(j, b):
        pltpu.make_async_copy(g_hbm.at[src_v.at[j]], rows[b], gsem[b]).wait()

    def scat_start(j, b):
        pltpu.async_copy(rows[b], acc.at[dst_v.at[j]], ssem[b], add=True)

    def scat_wait(j, b):
        pltpu.make_async_copy(rows[b], acc.at[dst_v.at[j]], ssem[b]).wait()

    for h in range(_NIDX):
        pltpu.sync_copy(src_hbm.at[wid, pl.ds(h * _CHH, _CHH)], src_v)
        pltpu.sync_copy(dst_hbm.at[wid, pl.ds(h * _CHH, _CHH)], dst_v)

        gather_start(0, 0)
        gather_start(1, 1)

        @pl.loop(0, _CHH, step=2)
        def _(j):
            gather_wait(j, 0)
            scat_start(j, 0)

            @pl.when(j + 2 < _CHH)
            def _():
                scat_wait(j, 0)
                gather_start(j + 2, 0)

            gather_wait(j + 1, 1)
            scat_start(j + 1, 1)

            @pl.when(j + 3 < _CHH)
            def _():
                scat_wait(j + 1, 1)
                gather_start(j + 3, 1)

        scat_wait(_CHH - 2, 0)
        scat_wait(_CHH - 1, 1)

    plsc.subcore_barrier()
    pltpu.sync_copy(acc.at[pl.ds(s * _ZPT, _ZPT), :],
                    out_hbm.at[c, pl.ds(s * _ZPT, _ZPT), :])


def _sc_mesh():
    return plsc.VectorSubcoreMesh(core_axis_name="c", subcore_axis_name="s",
                                  num_cores=_NC, num_subcores=_NS)


def _sc_degree(dst_r):
    return pl.kernel(
        _sc_degree_body,
        out_type=jax.ShapeDtypeStruct((_NC, _ACC_ROWS), jnp.float32),
        mesh=_sc_mesh(),
        scratch_types=[
            pltpu.VMEM((_CH, _CHUNK), jnp.int32),
            pltpu.VMEM((_CHUNK,), jnp.float32),
            pltpu.VMEM((_ZPT,), jnp.float32),
            pltpu.VMEM_SHARED((_ACC_ROWS,), jnp.float32),
        ],
    )(dst_r)


def _sc_scatter(g, src_r, dst_r):
    return pl.kernel(
        _sc_scatter_body,
        out_type=jax.ShapeDtypeStruct((_NC, _ACC_ROWS, _D), jnp.float32),
        mesh=_sc_mesh(),
        scratch_types=(
            [pltpu.VMEM((_CHH, _CHUNK), jnp.int32)] * 2
            + [pltpu.VMEM((_CHUNK, _D), jnp.float32)] * _NBUF
            + [pltpu.VMEM_SHARED((_ACC_ROWS, _D), jnp.float32)]
            + [pltpu.SemaphoreType.DMA] * (2 * _NBUF)
        ),
    )(g, src_r, dst_r)


_BR = 1000


def _tc1_body(x_ref, w_ref, da_ref, db_ref, g_ref, dis_ref):
    deg = da_ref[...] + db_ref[...] + 1.0
    dis = lax.rsqrt(deg)
    dis_ref[...] = dis
    g_ref[...] = dis * jnp.dot(x_ref[...], w_ref[...],
                               preferred_element_type=jnp.float32)


def _tc_mid_body(pa, pb, g, dis, b, gm, bt, mn, vr, w, out):
    conv = dis[...] * (pa[0] + pb[0] + g[...]) + b[...]
    h = (conv - mn[...]) * lax.rsqrt(vr[...] + _EPS) * gm[...] + bt[...]
    h = jnp.maximum(h, 0.0)
    out[...] = dis[...] * jnp.dot(h, w[...],
                                  preferred_element_type=jnp.float32)


def _tc_fin_body(pa, pb, g, dis, b, gm, bt, mn, vr, oh_ref,
                 h_ref, pool_ref):
    i = pl.program_id(0)
    conv = dis[...] * (pa[0] + pb[0] + g[...]) + b[...]
    h = (conv - mn[...]) * lax.rsqrt(vr[...] + _EPS) * gm[...] + bt[...]
    h = jnp.maximum(h, 0.0)
    h_ref[...] = h
    part = lax.dot_general(oh_ref[...], h, (((0,), (0,)), ((), ())),
                           preferred_element_type=jnp.float32)

    @pl.when(i == 0)
    def _():
        pool_ref[...] = part

    @pl.when(i != 0)
    def _():
        pool_ref[...] += part


def _rows_spec():
    return pl.BlockSpec((_BR, _D), lambda i: (i, 0))


def _col_spec():
    return pl.BlockSpec((_BR, 1), lambda i: (i, 0))


def _param_spec():
    return pl.BlockSpec((1, _D), lambda i: (0, 0))


def _full_spec():
    return pl.BlockSpec((_D, _D), lambda i: (0, 0))


def _tc1(x, W0, da, db):
    return pl.pallas_call(
        _tc1_body,
        grid=(_N // _BR,),
        in_specs=[_rows_spec(), _full_spec(), _col_spec(), _col_spec()],
        out_specs=[_rows_spec(), _col_spec()],
        out_shape=[jax.ShapeDtypeStruct((_N, _D), jnp.float32),
                   jax.ShapeDtypeStruct((_N, 1), jnp.float32)],
    )(x, W0, da, db)


def _part_spec(core):
    return pl.BlockSpec((1, _BR, _D), lambda i, c=core: (c, i, 0))


def _tc_mid(p, g, dis, b, gm, bt, mn, vr, W):
    return pl.pallas_call(
        _tc_mid_body,
        grid=(_N // _BR,),
        in_specs=[_part_spec(0), _part_spec(1), _rows_spec(), _col_spec(),
                  _param_spec(), _param_spec(), _param_spec(),
                  _param_spec(), _param_spec(), _full_spec()],
        out_specs=_rows_spec(),
        out_shape=jax.ShapeDtypeStruct((_N, _D), jnp.float32),
    )(p, p, g, dis, b, gm, bt, mn, vr, W)


def _tc_fin(p, g, dis, b, gm, bt, mn, vr, onehot):
    return pl.pallas_call(
        _tc_fin_body,
        grid=(_N // _BR,),
        in_specs=[_part_spec(0), _part_spec(1), _rows_spec(), _col_spec(),
                  _param_spec(), _param_spec(), _param_spec(),
                  _param_spec(), _param_spec(),
                  pl.BlockSpec((_BR, _G), lambda i: (i, 0))],
        out_specs=[_rows_spec(), pl.BlockSpec((_G, _D), lambda i: (0, 0))],
        out_shape=[jax.ShapeDtypeStruct((_N, _D), jnp.float32),
                   jax.ShapeDtypeStruct((_G, _D), jnp.float32)],
    )(p, p, g, dis, b, gm, bt, mn, vr, onehot)


def kernel(x, edge_index, batch, W0, b0, bn0_gamma, bn0_beta, bn0_mean,
           bn0_var, W1, b1, bn1_gamma, bn1_beta, bn1_mean, bn1_var):
    pad = _E_PAD - _E
    pad_ids = jnp.arange(pad, dtype=jnp.int32)
    src_r = jnp.concatenate(
        [edge_index[0], pad_ids % _N]).reshape(_NW, _CH, _CHUNK)
    dst_r = jnp.concatenate(
        [edge_index[1], _N + pad_ids % (_ACC_ROWS - _N)]).reshape(_NW, _CH, _CHUNK)

    degp = _sc_degree(dst_r)
    da = degp[0, :_N, None]
    db = degp[1, :_N, None]

    onehot = (batch[:, None] ==
              jnp.arange(_G, dtype=jnp.int32)[None, :]).astype(jnp.float32)

    g0, dis = _tc1(x, W0, da, db)
    p0 = _sc_scatter(g0, src_r, dst_r)
    g1 = _tc_mid(p0, g0, dis,
                 b0[None, :], bn0_gamma[None, :], bn0_beta[None, :],
                 bn0_mean[None, :], bn0_var[None, :], W1)
    p1 = _sc_scatter(g1, src_r, dst_r)
    h1, pool = _tc_fin(p1, g1, dis,
                       b1[None, :], bn1_gamma[None, :], bn1_beta[None, :],
                       bn1_mean[None, :], bn1_var[None, :], onehot)
    return h1, pool


# speedup vs baseline: 1.1646x; 1.0222x over previous
import jax
import jax.numpy as jnp
from jax import lax
from jax.experimental import pallas as pl
from jax.experimental.pallas import tpu as pltpu
from jax.experimental.pallas import tpu_sc as plsc

_N = 10000
_E = 320000
_D = 128
_G = 64
_EPS = 1e-5

_NC = 2
_NS = 16
_NW = _NC * _NS
_CHUNK = 128
_EPT = 10240
_CH = _EPT // _CHUNK
_NIDX = 2
_CHH = _CH // _NIDX
_NBUF = 2
_E_PAD = _NW * _EPT
_ACC_ROWS = 10240
_ZPT = _ACC_ROWS // _NS
_RPT = _N // _NS


def _sc_degree_body(e_hbm, out_hbm, dst_v, ones_v, zv, acc):
    c = lax.axis_index("c")
    s = lax.axis_index("s")
    wid = s * _NC + c

    @pl.loop(0, _ZPT // 16)
    def _(i):
        zv[pl.ds(i * 16, 16)] = jnp.zeros((16,), jnp.float32)

    @pl.loop(0, _CHUNK // 16)
    def _(i):
        ones_v[pl.ds(i * 16, 16)] = jnp.ones((16,), jnp.float32)

    pltpu.sync_copy(zv, acc.at[pl.ds(s * _ZPT, _ZPT)])
    plsc.subcore_barrier()

    pltpu.sync_copy(e_hbm.at[1, wid], dst_v)

    @pl.loop(0, _CH)
    def _(j):
        pltpu.sync_copy(ones_v, acc.at[dst_v.at[j]], add=True)

    plsc.subcore_barrier()
    pltpu.sync_copy(acc.at[pl.ds(s * _ZPT, _ZPT)],
                    out_hbm.at[c, pl.ds(s * _ZPT, _ZPT)])


def _sc_scatter_body(g_hbm, e_hbm, out_hbm,
                     src_v, dst_v, rows0, rows1, acc,
                     gsem0, gsem1, ssem0, ssem1):
    c = lax.axis_index("c")
    s = lax.axis_index("s")
    wid = s * _NC + c
    rows = (rows0, rows1)
    gsem = (gsem0, gsem1)
    ssem = (ssem0, ssem1)

    @pl.loop(0, _CHUNK)
    def _(r):
        for c8 in range(8):
            rows0[r, pl.ds(c8 * 16, 16)] = jnp.zeros((16,), jnp.float32)

    @pl.loop(0, _ZPT // _CHUNK)
    def _(i):
        pltpu.sync_copy(rows0, acc.at[pl.ds(s * _ZPT + i * _CHUNK, _CHUNK), :])

    plsc.subcore_barrier()

    def gather_start(j, b):
        pltpu.async_copy(g_hbm.at[src_v.at[j]], rows[b], gsem[b])

    def gather_wait(j, b):
        pltpu.make_async_copy(g_hbm.at[src_v.at[j]], rows[b], gsem[b]).wait()

    def scat_start(j, b):
        pltpu.async_copy(rows[b], acc.at[dst_v.at[j]], ssem[b], add=True)

    def scat_wait(j, b):
        pltpu.make_async_copy(rows[b], acc.at[dst_v.at[j]], ssem[b]).wait()

    for h in range(_NIDX):
        pltpu.sync_copy(e_hbm.at[0, wid, pl.ds(h * _CHH, _CHH)], src_v)
        pltpu.sync_copy(e_hbm.at[1, wid, pl.ds(h * _CHH, _CHH)], dst_v)

        gather_start(0, 0)
        gather_start(1, 1)

        @pl.loop(0, _CHH, step=2)
        def _(j):
            gather_wait(j, 0)
            scat_start(j, 0)

            @pl.when(j + 2 < _CHH)
            def _():
                scat_wait(j, 0)
                gather_start(j + 2, 0)

            gather_wait(j + 1, 1)
            scat_start(j + 1, 1)

            @pl.when(j + 3 < _CHH)
            def _():
                scat_wait(j + 1, 1)
                gather_start(j + 3, 1)

        scat_wait(_CHH - 2, 0)
        scat_wait(_CHH - 1, 1)

    plsc.subcore_barrier()
    pltpu.sync_copy(acc.at[pl.ds(s * _ZPT, _ZPT), :],
                    out_hbm.at[c, pl.ds(s * _ZPT, _ZPT), :])


def _sc_mesh():
    return plsc.VectorSubcoreMesh(core_axis_name="c", subcore_axis_name="s",
                                  num_cores=_NC, num_subcores=_NS)


def _sc_degree(er):
    return pl.kernel(
        _sc_degree_body,
        out_type=jax.ShapeDtypeStruct((_NC, _ACC_ROWS), jnp.float32),
        mesh=_sc_mesh(),
        scratch_types=[
            pltpu.VMEM((_CH, _CHUNK), jnp.int32),
            pltpu.VMEM((_CHUNK,), jnp.float32),
            pltpu.VMEM((_ZPT,), jnp.float32),
            pltpu.VMEM_SHARED((_ACC_ROWS,), jnp.float32),
        ],
    )(er)


def _sc_scatter(g, er):
    return pl.kernel(
        _sc_scatter_body,
        out_type=jax.ShapeDtypeStruct((_NC, _ACC_ROWS, _D), jnp.float32),
        mesh=_sc_mesh(),
        scratch_types=(
            [pltpu.VMEM((_CHH, _CHUNK), jnp.int32)] * 2
            + [pltpu.VMEM((_CHUNK, _D), jnp.float32)] * _NBUF
            + [pltpu.VMEM_SHARED((_ACC_ROWS, _D), jnp.float32)]
            + [pltpu.SemaphoreType.DMA] * (2 * _NBUF)
        ),
    )(g, er)


_BR = 1000


def _tc1_body(x_ref, w_ref, da_ref, db_ref, g_ref, dis_ref):
    deg = da_ref[...] + db_ref[...] + 1.0
    dis = lax.rsqrt(deg)
    dis_ref[...] = dis
    g_ref[...] = dis * jnp.dot(x_ref[...], w_ref[...],
                               preferred_element_type=jnp.float32)


def _tc_mid_body(pa, pb, g, dis, b, gm, bt, mn, vr, w, out):
    conv = dis[...] * (pa[0] + pb[0] + g[...]) + b[...]
    h = (conv - mn[...]) * lax.rsqrt(vr[...] + _EPS) * gm[...] + bt[...]
    h = jnp.maximum(h, 0.0)
    out[...] = dis[...] * jnp.dot(h, w[...],
                                  preferred_element_type=jnp.float32)


def _tc_fin_body(pa, pb, g, dis, b, gm, bt, mn, vr, oh_ref,
                 h_ref, pool_ref):
    i = pl.program_id(0)
    conv = dis[...] * (pa[0] + pb[0] + g[...]) + b[...]
    h = (conv - mn[...]) * lax.rsqrt(vr[...] + _EPS) * gm[...] + bt[...]
    h = jnp.maximum(h, 0.0)
    h_ref[...] = h
    part = lax.dot_general(oh_ref[...], h, (((0,), (0,)), ((), ())),
                           preferred_element_type=jnp.float32)

    @pl.when(i == 0)
    def _():
        pool_ref[...] = part

    @pl.when(i != 0)
    def _():
        pool_ref[...] += part


def _rows_spec():
    return pl.BlockSpec((_BR, _D), lambda i: (i, 0))


def _col_spec():
    return pl.BlockSpec((_BR, 1), lambda i: (i, 0))


def _param_spec():
    return pl.BlockSpec((1, _D), lambda i: (0, 0))


def _full_spec():
    return pl.BlockSpec((_D, _D), lambda i: (0, 0))


def _tc1(x, W0, da, db):
    return pl.pallas_call(
        _tc1_body,
        grid=(_N // _BR,),
        in_specs=[_rows_spec(), _full_spec(), _col_spec(), _col_spec()],
        out_specs=[_rows_spec(), _col_spec()],
        out_shape=[jax.ShapeDtypeStruct((_N, _D), jnp.float32),
                   jax.ShapeDtypeStruct((_N, 1), jnp.float32)],
    )(x, W0, da, db)


def _part_spec(core):
    return pl.BlockSpec((1, _BR, _D), lambda i, c=core: (c, i, 0))


def _tc_mid(p, g, dis, b, gm, bt, mn, vr, W):
    return pl.pallas_call(
        _tc_mid_body,
        grid=(_N // _BR,),
        in_specs=[_part_spec(0), _part_spec(1), _rows_spec(), _col_spec(),
                  _param_spec(), _param_spec(), _param_spec(),
                  _param_spec(), _param_spec(), _full_spec()],
        out_specs=_rows_spec(),
        out_shape=jax.ShapeDtypeStruct((_N, _D), jnp.float32),
    )(p, p, g, dis, b, gm, bt, mn, vr, W)


def _tc_fin(p, g, dis, b, gm, bt, mn, vr, onehot):
    return pl.pallas_call(
        _tc_fin_body,
        grid=(_N // _BR,),
        in_specs=[_part_spec(0), _part_spec(1), _rows_spec(), _col_spec(),
                  _param_spec(), _param_spec(), _param_spec(),
                  _param_spec(), _param_spec(),
                  pl.BlockSpec((_BR, _G), lambda i: (i, 0))],
        out_specs=[_rows_spec(), pl.BlockSpec((_G, _D), lambda i: (0, 0))],
        out_shape=[jax.ShapeDtypeStruct((_N, _D), jnp.float32),
                   jax.ShapeDtypeStruct((_G, _D), jnp.float32)],
    )(p, p, g, dis, b, gm, bt, mn, vr, onehot)


def kernel(x, edge_index, batch, W0, b0, bn0_gamma, bn0_beta, bn0_mean,
           bn0_var, W1, b1, bn1_gamma, bn1_beta, bn1_mean, bn1_var):
    pad = _E_PAD - _E
    pad_ids = jnp.arange(pad, dtype=jnp.int32)
    pads = jnp.stack([pad_ids % _N, _N + pad_ids % (_ACC_ROWS - _N)])
    er = jnp.concatenate([edge_index, pads], axis=1).reshape(
        2, _NW, _CH, _CHUNK)

    degp = _sc_degree(er)
    da = degp[0, :_N, None]
    db = degp[1, :_N, None]

    onehot = (batch[:, None] ==
              jnp.arange(_G, dtype=jnp.int32)[None, :]).astype(jnp.float32)

    g0, dis = _tc1(x, W0, da, db)
    p0 = _sc_scatter(g0, er)
    g1 = _tc_mid(p0, g0, dis,
                 b0[None, :], bn0_gamma[None, :], bn0_beta[None, :],
                 bn0_mean[None, :], bn0_var[None, :], W1)
    p1 = _sc_scatter(g1, er)
    h1, pool = _tc_fin(p1, g1, dis,
                       b1[None, :], bn1_gamma[None, :], bn1_beta[None, :],
                       bn1_mean[None, :], bn1_var[None, :], onehot)
    return h1, pool
